# baseline (speedup 1.0000x reference)
import jax
import jax.numpy as jnp
import numpy as np
from jax.experimental import pallas as pl


_B, _N, _CIN = 256, 511, 318
_NP1 = _N + 1
_S = 4


def _ln_relu(out, o):
    n = o * _NP1
    s1 = jnp.sum(out, keepdims=True)
    s2 = jnp.sum(out * out, keepdims=True)
    m = s1 / n
    var = (s2 - m * s1) / (n - 1)
    return jnp.maximum((out - m) / (jnp.sqrt(var) + 1e-5), 0.0)


def _mm(a, b, out_dtype):
    return jnp.dot(a, b, preferred_element_type=out_dtype)


def _fused(idx_ref, trees_ref, wenc_ref, benc_ref, w1_ref, b1_ref,
           w2_ref, b2_ref, w3_ref, b3_ref, whead_ref, bhead_ref,
           lat_ref, cost_ref):
    f32, bf16 = jnp.float32, jnp.bfloat16
    f8 = jnp.float8_e4m3fn
    iota_sub = jax.lax.broadcasted_iota(jnp.int32, (_NP1, _NP1), 0)

    s_mats = [[(iota_sub == idx_ref[s, k][None, :]).astype(f8)
               for k in range(3)] for s in range(_S)]

    x8 = [(_mm(wenc_ref[...], trees_ref[s], f32) + benc_ref[...]
           ).astype(f8) for s in range(_S)]

    w1 = w1_ref[...]
    nxt = []
    for s in range(_S):
        acc = b1_ref[...]
        for k in range(3):
            g = _mm(x8[s], s_mats[s][k], f32).astype(bf16)
            acc = acc + _mm(w1[k * 512:(k + 1) * 512], g, f32)
        nxt.append(_ln_relu(acc, 512).astype(bf16))
    xb = nxt

    x_last = None
    for w_ref, b_ref, o in ((w2_ref, b2_ref, 256), (w3_ref, b3_ref, 128)):
        nxt, x_last = [], []
        for s in range(_S):
            y = _mm(w_ref[...], xb[s], f32).astype(f8)
            acc = b_ref[...]
            for k in range(3):
                acc = acc + _mm(y[k * o:(k + 1) * o], s_mats[s][k], f32)
            x = _ln_relu(acc, o)
            x_last.append(x)
            nxt.append(x.astype(bf16))
        xb = nxt

    for s in range(_S):
        pooled = jnp.max(x_last[s], axis=1, keepdims=True)
        z = jnp.sum(pooled * whead_ref[...], axis=0, keepdims=True)
        sig = jax.nn.sigmoid(z + bhead_ref[...])
        lat_ref[s] = jnp.broadcast_to(sig[:, 0:1], (1, 128))
        cost_ref[s] = jnp.broadcast_to(sig[:, 1:2], (1, 128))


def kernel(trees, indexes, W_enc, b_enc, Wc1, bc1, Wc2, bc2, Wc3, bc3,
           W_lat, b_lat, W_cost, b_cost):
    B = trees.shape[0]
    idx3 = indexes[:, :, 0].astype(jnp.int32).reshape(B, _N, 3)
    idx3 = jnp.transpose(idx3, (0, 2, 1))
    idx_sh = jnp.concatenate(
        [jnp.full((B, 3, 1), -1, jnp.int32), idx3], axis=2)

    def cat(w):
        o, c, _ = w.shape
        return jnp.moveaxis(w, 2, 0).reshape(3 * o, c).astype(jnp.bfloat16)

    wenc_t = W_enc.T.astype(jnp.bfloat16)
    trees_b = trees.astype(jnp.bfloat16)
    w1, w2, w3 = cat(Wc1), cat(Wc2), cat(Wc3)
    whead = jnp.concatenate([W_lat, W_cost], axis=1)
    bhead = jnp.concatenate([b_lat, b_cost])[None, :]

    maskrow = (jnp.arange(_NP1) >= 1).astype(jnp.float32)[None, :]
    benc_map = jnp.broadcast_to(b_enc[:, None], (109, _NP1))
    b1_map = bc1[:, None] * maskrow
    b2_map = bc2[:, None] * maskrow
    b3_map = bc3[:, None] * maskrow

    def whole(a):
        return pl.BlockSpec(a.shape, lambda b: (0,) * a.ndim)

    consts = [wenc_t, benc_map, w1, b1_map, w2, b2_map, w3, b3_map,
              whead, bhead]

    def call(idx_sh, trees_b, *consts):
        b_loc = trees_b.shape[0]
        return pl.pallas_call(
            _fused,
            grid=(b_loc // _S,),
            in_specs=[
                pl.BlockSpec((_S, 3, _NP1), lambda b: (b, 0, 0)),
                pl.BlockSpec((_S, _CIN, _NP1), lambda b: (b, 0, 0)),
                *[whole(a) for a in consts],
            ],
            out_specs=[
                pl.BlockSpec((_S, 1, 128), lambda b: (b, 0, 0)),
                pl.BlockSpec((_S, 1, 128), lambda b: (b, 0, 0)),
            ],
            out_shape=[
                jax.ShapeDtypeStruct((b_loc, 1, 128), jnp.float32),
                jax.ShapeDtypeStruct((b_loc, 1, 128), jnp.float32),
            ],
        )(idx_sh, trees_b, *consts)

    devs = jax.devices()
    n_dev = len(devs)
    if n_dev > 1 and B % (n_dev * _S) == 0:
        mesh = jax.sharding.Mesh(np.array(devs), ("b",))
        p_b = jax.sharding.PartitionSpec("b")
        p_r = jax.sharding.PartitionSpec()
        call = jax.shard_map(call, mesh=mesh,
                             in_specs=(p_b, p_b) + (p_r,) * len(consts),
                             out_specs=(p_b, p_b), check_vma=False)
    lat, cost = call(idx_sh, trees_b, *consts)
    return lat[:, 0, :1], cost[:, 0, :1]


# speedup vs baseline: 1.3118x; 1.3118x over previous
import jax
import jax.numpy as jnp
from jax.experimental import pallas as pl


_B, _N, _CIN = 256, 511, 318
_NP1 = _N + 1
_S = 4


def _ln_relu(out, o):
    n = o * _NP1
    s1 = jnp.sum(out, keepdims=True)
    s2 = jnp.sum(out * out, keepdims=True)
    m = s1 / n
    var = (s2 - m * s1) / (n - 1)
    return jnp.maximum((out - m) / (jnp.sqrt(var) + 1e-5), 0.0)


def _mm(a, b, out_dtype):
    return jnp.dot(a, b, preferred_element_type=out_dtype)


def _fused(idx_ref, trees_ref, wenc_ref, benc_ref, w1_ref, b1_ref,
           w2_ref, b2_ref, w3_ref, b3_ref, whead_ref, bhead_ref,
           lat_ref, cost_ref):
    f32, bf16 = jnp.float32, jnp.bfloat16
    f8 = jnp.float8_e4m3fn
    iota_sub = jax.lax.broadcasted_iota(jnp.int32, (_NP1, _NP1), 0)

    s_mats = [[(iota_sub == idx_ref[s, k][None, :]).astype(f8)
               for k in range(3)] for s in range(_S)]

    x8 = [(_mm(wenc_ref[...], trees_ref[s], f32) + benc_ref[...]
           ).astype(f8) for s in range(_S)]

    w1 = w1_ref[...]
    nxt = []
    for s in range(_S):
        acc = b1_ref[...]
        for k in range(3):
            g = _mm(x8[s], s_mats[s][k], f32).astype(bf16)
            acc = acc + _mm(w1[k * 512:(k + 1) * 512], g, f32)
        nxt.append(_ln_relu(acc, 512).astype(bf16))
    xb = nxt

    x_last = None
    for w_ref, b_ref, o in ((w2_ref, b2_ref, 256), (w3_ref, b3_ref, 128)):
        nxt, x_last = [], []
        for s in range(_S):
            y = _mm(w_ref[...], xb[s], f32).astype(f8)
            acc = b_ref[...]
            for k in range(3):
                acc = acc + _mm(y[k * o:(k + 1) * o], s_mats[s][k], f32)
            x = _ln_relu(acc, o)
            x_last.append(x)
            nxt.append(x.astype(bf16))
        xb = nxt

    for s in range(_S):
        pooled = jnp.max(x_last[s], axis=1, keepdims=True)
        z = jnp.sum(pooled * whead_ref[...], axis=0, keepdims=True)
        sig = jax.nn.sigmoid(z + bhead_ref[...])
        lat_ref[s] = jnp.broadcast_to(sig[:, 0:1], (1, 128))
        cost_ref[s] = jnp.broadcast_to(sig[:, 1:2], (1, 128))


def kernel(trees, indexes, W_enc, b_enc, Wc1, bc1, Wc2, bc2, Wc3, bc3,
           W_lat, b_lat, W_cost, b_cost):
    B = trees.shape[0]
    idx3 = indexes[:, :, 0].astype(jnp.int32).reshape(B, _N, 3)
    idx3 = jnp.transpose(idx3, (0, 2, 1))
    idx_sh = jnp.concatenate(
        [jnp.full((B, 3, 1), -1, jnp.int32), idx3], axis=2)

    def cat(w):
        o, c, _ = w.shape
        return jnp.moveaxis(w, 2, 0).reshape(3 * o, c).astype(jnp.bfloat16)

    wenc_t = W_enc.T.astype(jnp.bfloat16)
    trees_b = trees.astype(jnp.bfloat16)
    w1, w2, w3 = cat(Wc1), cat(Wc2), cat(Wc3)
    whead = jnp.concatenate([W_lat, W_cost], axis=1)
    bhead = jnp.concatenate([b_lat, b_cost])[None, :]

    maskrow = (jnp.arange(_NP1) >= 1).astype(jnp.float32)[None, :]
    benc_map = jnp.broadcast_to(b_enc[:, None], (109, _NP1))
    b1_map = bc1[:, None] * maskrow
    b2_map = bc2[:, None] * maskrow
    b3_map = bc3[:, None] * maskrow

    def whole(a):
        return pl.BlockSpec(a.shape, lambda b: (0,) * a.ndim)

    consts = [wenc_t, benc_map, w1, b1_map, w2, b2_map, w3, b3_map,
              whead, bhead]

    def call(idx_sh, trees_b, *consts):
        b_loc = trees_b.shape[0]
        return pl.pallas_call(
            _fused,
            grid=(b_loc // _S,),
            in_specs=[
                pl.BlockSpec((_S, 3, _NP1), lambda b: (b, 0, 0)),
                pl.BlockSpec((_S, _CIN, _NP1), lambda b: (b, 0, 0)),
                *[whole(a) for a in consts],
            ],
            out_specs=[
                pl.BlockSpec((_S, 1, 128), lambda b: (b, 0, 0)),
                pl.BlockSpec((_S, 1, 128), lambda b: (b, 0, 0)),
            ],
            out_shape=[
                jax.ShapeDtypeStruct((b_loc, 1, 128), jnp.float32),
                jax.ShapeDtypeStruct((b_loc, 1, 128), jnp.float32),
            ],
        )(idx_sh, trees_b, *consts)

    lat, cost = call(idx_sh, trees_b, *consts)
    return lat[:, 0, :1], cost[:, 0, :1]
